# baseline (speedup 1.0000x reference)
import functools

import jax
import jax.numpy as jnp
from jax import lax
from jax.experimental import pallas as pl
from jax.experimental.pallas import tpu as pltpu
from jax.experimental.pallas import tpu_sc as plsc

N_EVENTS = 6000
N_OBJECTS = 4000
N_NODES = 10000
N_HEDGES = 2000
N_INC = 320000
D = 128

NC = 2
NS = 16
NW = NC * NS
PER_TILE = N_INC // NW
K = 125
KP = 128
CH = PER_TILE // K
SEG = 5
SCH = CH // SEG

NPAD = 10240
HPAD = 2048

_mesh = lambda: plsc.VectorSubcoreMesh(
    core_axis_name="c", subcore_axis_name="s", num_cores=NC, num_subcores=NS)

_sc_params = lambda: pltpu.CompilerParams(needs_layout_passes=False)


def _counts_body(gn, gh, outn, outh, gn_v, gh_v, ncnt_v, ecnt_v):
  c = lax.axis_index("c")
  s = lax.axis_index("s")
  wid = c * NS + s
  pltpu.sync_copy(gn.at[wid], gn_v)
  pltpu.sync_copy(gh.at[wid], gh_v)
  zeros16 = jnp.zeros((16,), jnp.float32)

  def zn(i, _):
    ncnt_v[pl.ds(i * 16, 16)] = zeros16
    return 0
  lax.fori_loop(0, NPAD // 16, zn, 0)

  def zh(i, _):
    ecnt_v[pl.ds(i * 16, 16)] = zeros16
    return 0
  lax.fori_loop(0, HPAD // 16, zh, 0)

  ones16 = jnp.ones((16,), jnp.float32)

  def crow(j, _):
    for g in range(KP // 16):
      vn = gn_v[j, pl.ds(g * 16, 16)]
      plsc.addupdate_scatter(ncnt_v, [vn], ones16)
      vh = gh_v[j, pl.ds(g * 16, 16)]
      plsc.addupdate_scatter(ecnt_v, [vh], ones16)
    return 0
  lax.fori_loop(0, CH, crow, 0)

  pltpu.sync_copy(ncnt_v, outn.at[wid])
  pltpu.sync_copy(ecnt_v, outh.at[wid])


@jax.jit
def _sc_counts(gn, gh):
  return pl.kernel(
      _counts_body,
      out_type=(jax.ShapeDtypeStruct((NW, NPAD), jnp.float32),
                jax.ShapeDtypeStruct((NW, HPAD), jnp.float32)),
      mesh=_mesh(),
      compiler_params=_sc_params(),
      scratch_types=[
          pltpu.VMEM((CH, KP), jnp.int32),
          pltpu.VMEM((CH, KP), jnp.int32),
          pltpu.VMEM((NPAD,), jnp.float32),
          pltpu.VMEM((HPAD,), jnp.float32),
      ],
  )(gn, gh)


def _seg_body(opad, table, gidx, sidx, zbuf, out,
              gidx_v, sidx_v, buf0, buf1, accum, sem0, sem1):
  c = lax.axis_index("c")
  s = lax.axis_index("s")
  wid = c * NS + s
  zr = opad // NS
  pltpu.sync_copy(zbuf, accum.at[pl.ds(s * zr, zr)])
  plsc.subcore_barrier()

  def seg(t, _):
    pltpu.sync_copy(gidx.at[wid, pl.ds(t * SCH, SCH)], gidx_v)
    pltpu.sync_copy(sidx.at[wid, pl.ds(t * SCH, SCH)], sidx_v)
    pltpu.async_copy(table.at[gidx_v.at[0]], buf0, sem0)

    def pair(i, _):
      j = 2 * i
      pltpu.make_async_copy(table.at[gidx_v.at[j]], buf0, sem0).wait()
      pltpu.async_copy(table.at[gidx_v.at[j + 1]], buf1, sem1)
      pltpu.sync_copy(buf0, accum.at[sidx_v.at[j]], add=True)
      pltpu.make_async_copy(table.at[gidx_v.at[j + 1]], buf1, sem1).wait()

      @pl.when(j + 2 < SCH)
      def _():
        pltpu.async_copy(table.at[gidx_v.at[j + 2]], buf0, sem0)

      pltpu.sync_copy(buf1, accum.at[sidx_v.at[j + 1]], add=True)
      return 0
    lax.fori_loop(0, SCH // 2, pair, 0)
    return 0
  lax.fori_loop(0, SEG, seg, 0)

  plsc.subcore_barrier()
  pltpu.sync_copy(accum.at[pl.ds(s * zr, zr)], out.at[c, pl.ds(s * zr, zr)])


@functools.partial(jax.jit, static_argnames=("opad",))
def _sc_seg(table, gidx, sidx, zbuf, opad):
  return pl.kernel(
      functools.partial(_seg_body, opad),
      out_type=jax.ShapeDtypeStruct((NC, opad, D), jnp.float32),
      mesh=_mesh(),
      compiler_params=_sc_params(),
      scratch_types=[
          pltpu.VMEM((SCH, KP), jnp.int32),
          pltpu.VMEM((SCH, KP), jnp.int32),
          pltpu.VMEM((KP, D), jnp.float32),
          pltpu.VMEM((KP, D), jnp.float32),
          pltpu.VMEM_SHARED((opad, D), jnp.float32),
          pltpu.SemaphoreType.DMA,
          pltpu.SemaphoreType.DMA,
      ],
  )(table, gidx, sidx, zbuf)


def _gather_body(table, idx, out, idx_v, rows_v, sem):
  c = lax.axis_index("c")
  s = lax.axis_index("s")
  wid = c * NS + s
  bpw = 2048 // NW
  base = wid * bpw
  pltpu.sync_copy(idx.at[pl.ds(base, bpw)], idx_v)
  pltpu.async_copy(table.at[idx_v], rows_v, sem).wait()
  pltpu.sync_copy(rows_v, out.at[pl.ds(base, bpw)])


@jax.jit
def _sc_gather(table, idx):
  bpw = 2048 // NW
  return pl.kernel(
      _gather_body,
      out_type=jax.ShapeDtypeStruct((2048, D), jnp.float32),
      mesh=_mesh(),
      compiler_params=_sc_params(),
      scratch_types=[
          pltpu.VMEM((bpw,), jnp.int32),
          pltpu.VMEM((bpw, D), jnp.float32),
          pltpu.SemaphoreType.DMA,
      ],
  )(table, idx)


def _proj_body(x_ref, w_ref, b_ref, g_ref, be_ref, o_ref):
  y = jnp.dot(x_ref[...], w_ref[...], preferred_element_type=jnp.float32)
  y = y + b_ref[...]
  y = jnp.where(y >= 0, y, 0.2 * y)
  m = jnp.mean(y, axis=-1, keepdims=True)
  v = jnp.mean((y - m) ** 2, axis=-1, keepdims=True)
  o_ref[...] = (y - m) / jnp.sqrt(v + 1e-5) * g_ref[...] + be_ref[...]


@jax.jit
def _proj(x, w, b, g, be):
  n = x.shape[0]
  rb = 1000
  grid = n // rb
  return pl.pallas_call(
      _proj_body,
      grid=(grid,),
      in_specs=[
          pl.BlockSpec((rb, D), lambda i: (i, 0)),
          pl.BlockSpec((D, D), lambda i: (0, 0)),
          pl.BlockSpec((1, D), lambda i: (0, 0)),
          pl.BlockSpec((1, D), lambda i: (0, 0)),
          pl.BlockSpec((1, D), lambda i: (0, 0)),
      ],
      out_specs=pl.BlockSpec((rb, D), lambda i: (i, 0)),
      out_shape=jax.ShapeDtypeStruct((n, D), jnp.float32),
  )(x, w, b.reshape(1, D), g.reshape(1, D), be.reshape(1, D))


def _combine_body(p_ref, c_ref, o_ref):
  cnt = jnp.maximum(jnp.sum(c_ref[...], axis=0), 1.0)
  o_ref[...] = (p_ref[0] + p_ref[1]) * (1.0 / cnt)[:, None]


@jax.jit
def _combine(parts, cparts):
  rb = 256
  grid = HPAD // rb
  return pl.pallas_call(
      _combine_body,
      grid=(grid,),
      in_specs=[
          pl.BlockSpec((NC, rb, D), lambda i: (0, i, 0)),
          pl.BlockSpec((NW, rb), lambda i: (0, i)),
      ],
      out_specs=pl.BlockSpec((rb, D), lambda i: (i, 0)),
      out_shape=jax.ShapeDtypeStruct((HPAD, D), jnp.float32),
  )(parts, cparts)


def _update_body(p_ref, c_ref, x_ref, w_ref, b_ref, o_ref):
  cnt = jnp.maximum(jnp.sum(c_ref[...], axis=0), 1.0)
  nf = (p_ref[0] + p_ref[1]) * (1.0 / cnt)[:, None]
  y = jnp.dot(nf, w_ref[...], preferred_element_type=jnp.float32) + b_ref[...]
  o_ref[...] = jnp.maximum(y, 0.0) + x_ref[...]


@jax.jit
def _update(parts, cparts, xres, w, b):
  rb = 1024
  grid = NPAD // rb
  return pl.pallas_call(
      _update_body,
      grid=(grid,),
      in_specs=[
          pl.BlockSpec((NC, rb, D), lambda i: (0, i, 0)),
          pl.BlockSpec((NW, rb), lambda i: (0, i)),
          pl.BlockSpec((rb, D), lambda i: (i, 0)),
          pl.BlockSpec((D, D), lambda i: (0, 0)),
          pl.BlockSpec((1, D), lambda i: (0, 0)),
      ],
      out_specs=pl.BlockSpec((rb, D), lambda i: (i, 0)),
      out_shape=jax.ShapeDtypeStruct((N_NODES, D), jnp.float32),
  )(parts, cparts, xres, w, b.reshape(1, D))


def _fusion_body(ev_ref, ob_ref, w1_ref, w2_ref, b_ref, o_ref):
  ev = ev_ref[...]
  ob = ob_ref[...]
  z = (jnp.dot(ob, w1_ref[...], preferred_element_type=jnp.float32)
       + jnp.dot(ev, w2_ref[...], preferred_element_type=jnp.float32)
       + b_ref[...])
  g = jax.nn.sigmoid(z)
  o_ref[...] = g * ob + (1.0 - g) * ev


@jax.jit
def _fusion(ev, ob, w1, w2, b):
  n = ev.shape[0]
  return pl.pallas_call(
      _fusion_body,
      grid=(1,),
      in_specs=[
          pl.BlockSpec((n, D), lambda i: (0, 0)),
          pl.BlockSpec((n, D), lambda i: (0, 0)),
          pl.BlockSpec((D, D), lambda i: (0, 0)),
          pl.BlockSpec((D, D), lambda i: (0, 0)),
          pl.BlockSpec((1, D), lambda i: (0, 0)),
      ],
      out_specs=pl.BlockSpec((n, D), lambda i: (0, 0)),
      out_shape=jax.ShapeDtypeStruct((n, D), jnp.float32),
  )(ev, ob, w1, w2, b.reshape(1, D))


def kernel(object_X, event_X, W_ev, b_ev, g_ev, be_ev, W_ob, b_ob, g_ob, be_ob,
           W1, b1, W2, b2, Wg, bg, node_idx, hedge_idx, main_object, event_sel):
  ev = _proj(event_X, W_ev, b_ev, g_ev, be_ev)
  ob = _proj(object_X, W_ob, b_ob, g_ob, be_ob)
  X = jnp.concatenate([ev, ob], axis=0)

  pw = ((0, 0), (0, 0), (0, KP - K))
  gnr = node_idx.reshape(NW, CH, K)
  ghr = hedge_idx.reshape(NW, CH, K)
  gn_g = jnp.pad(gnr, pw)
  gh_g = jnp.pad(ghr, pw)
  gn_s = jnp.pad(gnr, pw, constant_values=NPAD - 1)
  gh_s = jnp.pad(ghr, pw, constant_values=HPAD - 1)
  ncp, ecp = _sc_counts(gn_s, gh_s)

  zn = jnp.zeros((NPAD // NS, D), jnp.float32)
  zh = jnp.zeros((HPAD // NS, D), jnp.float32)

  e1p = _sc_seg(X, gn_g, gh_s, zh, opad=HPAD)
  ef1 = _combine(e1p, ecp)
  n1p = _sc_seg(ef1, gh_g, gn_s, zn, opad=NPAD)
  H1 = _update(n1p, ncp, X, W1, b1)

  e2p = _sc_seg(H1, gn_g, gh_s, zh, opad=HPAD)
  ef2 = _combine(e2p, ecp)
  n2p = _sc_seg(ef2, gh_g, gn_s, zn, opad=NPAD)
  H2 = _update(n2p, ncp, H1, W2, b2)

  sel = jnp.concatenate([event_sel, main_object + N_EVENTS], axis=0)
  rows = _sc_gather(H2, sel)
  return _fusion(rows[:1024], rows[1024:], Wg[:D], Wg[D:], bg)


# speedup vs baseline: 3.0178x; 3.0178x over previous
import functools

import jax
import jax.numpy as jnp
from jax import lax
from jax.experimental import pallas as pl
from jax.experimental.pallas import tpu as pltpu
from jax.experimental.pallas import tpu_sc as plsc

N_EVENTS = 6000
N_OBJECTS = 4000
N_NODES = 10000
N_HEDGES = 2000
N_INC = 320000
D = 128

NC = 2
NS = 16
NW = NC * NS

NP = 10240
CP = 2048
RB = 320
SW = NP // (NC * RB)
ROWS_SC = RB * SW
IPT = N_INC // NS
RN = RB * CP
TILE_ELS = RN // NS

_mesh = lambda: plsc.VectorSubcoreMesh(
    core_axis_name="c", subcore_axis_name="s", num_cores=NC, num_subcores=NS)

_sc_params = lambda: pltpu.CompilerParams(needs_layout_passes=False)


ZB = 8192
DUMP = 2048


def _build_body(flat, out, flat_v, off_v, ones_v, zero_v, accum):
  c = lax.axis_index("c")
  s = lax.axis_index("s")
  pltpu.sync_copy(flat.at[s], flat_v)
  zeros16 = jnp.zeros((16,), jnp.float32)
  ones16 = jnp.ones((16,), jnp.float32)

  def of(i, _):
    ones_v[pl.ds(i * 16, 16)] = ones16
    return 0
  lax.fori_loop(0, IPT // 16, of, 0)

  def zf(i, _):
    zero_v[pl.ds(i * 16, 16)] = zeros16
    return 0
  lax.fori_loop(0, ZB // 16, zf, 0)

  def sweep(k, _):
    base = (c * ROWS_SC + k * RB) * CP

    def z(i, _):
      pltpu.sync_copy(zero_v, accum.at[pl.ds(s * TILE_ELS + i * ZB, ZB)])
      return 0
    lax.fori_loop(0, TILE_ELS // ZB, z, 0)

    def chunk(i, _):
      fv = flat_v[pl.ds(i * 16, 16)]
      off = fv - base
      inb = (off >= 0) & (off < RN)
      off_v[pl.ds(i * 16, 16)] = jnp.where(
          inb, off, RN + (fv & (DUMP - 1)))
      return 0
    lax.fori_loop(0, IPT // 16, chunk, 0)
    plsc.subcore_barrier()

    pltpu.sync_copy(ones_v, accum.at[off_v], add=True)
    plsc.subcore_barrier()

    row0 = c * ROWS_SC + k * RB
    pltpu.sync_copy(accum.at[pl.ds(s * TILE_ELS, TILE_ELS)],
                    out.at[pl.ds(row0 * CP + s * TILE_ELS, TILE_ELS)])
    plsc.subcore_barrier()
    return 0
  lax.fori_loop(0, SW, sweep, 0)


@jax.jit
def _sc_build(flat):
  return pl.kernel(
      _build_body,
      out_type=jax.ShapeDtypeStruct((NP * CP,), jnp.float32),
      mesh=_mesh(),
      compiler_params=_sc_params(),
      scratch_types=[
          pltpu.VMEM((IPT,), jnp.int32),
          pltpu.VMEM((IPT,), jnp.int32),
          pltpu.VMEM((IPT,), jnp.float32),
          pltpu.VMEM((ZB,), jnp.float32),
          pltpu.VMEM_SHARED((RN + DUMP,), jnp.float32),
      ],
  )(flat)


def _gather_body(table, idx, out, idx_v, rows_v, sem):
  c = lax.axis_index("c")
  s = lax.axis_index("s")
  wid = c * NS + s
  bpw = 2048 // NW
  base = wid * bpw
  pltpu.sync_copy(idx.at[pl.ds(base, bpw)], idx_v)
  pltpu.async_copy(table.at[idx_v], rows_v, sem).wait()
  pltpu.sync_copy(rows_v, out.at[pl.ds(base, bpw)])


@jax.jit
def _sc_gather(table, idx):
  bpw = 2048 // NW
  return pl.kernel(
      _gather_body,
      out_type=jax.ShapeDtypeStruct((2048, D), jnp.float32),
      mesh=_mesh(),
      compiler_params=_sc_params(),
      scratch_types=[
          pltpu.VMEM((bpw,), jnp.int32),
          pltpu.VMEM((bpw, D), jnp.float32),
          pltpu.SemaphoreType.DMA,
      ],
  )(table, idx)


def _proj_body(x_ref, w_ref, b_ref, g_ref, be_ref, o_ref):
  y = jnp.dot(x_ref[...], w_ref[...], preferred_element_type=jnp.float32)
  y = y + b_ref[...]
  y = jnp.where(y >= 0, y, 0.2 * y)
  m = jnp.mean(y, axis=-1, keepdims=True)
  v = jnp.mean((y - m) ** 2, axis=-1, keepdims=True)
  o_ref[...] = (y - m) / jnp.sqrt(v + 1e-5) * g_ref[...] + be_ref[...]


@jax.jit
def _proj(x, w, b, g, be):
  n = x.shape[0]
  rb = 1000
  grid = n // rb
  return pl.pallas_call(
      _proj_body,
      grid=(grid,),
      in_specs=[
          pl.BlockSpec((rb, D), lambda i: (i, 0)),
          pl.BlockSpec((D, D), lambda i: (0, 0)),
          pl.BlockSpec((1, D), lambda i: (0, 0)),
          pl.BlockSpec((1, D), lambda i: (0, 0)),
          pl.BlockSpec((1, D), lambda i: (0, 0)),
      ],
      out_specs=pl.BlockSpec((rb, D), lambda i: (i, 0)),
      out_shape=jax.ShapeDtypeStruct((n, D), jnp.float32),
  )(x, w, b.reshape(1, D), g.reshape(1, D), be.reshape(1, D))


def _edge_body(c_ref, xa_ref, o_ref, acc_ref):
  i = pl.program_id(0)

  @pl.when(i == 0)
  def _():
    acc_ref[...] = jnp.zeros_like(acc_ref)

  acc_ref[...] += lax.dot_general(
      c_ref[...], xa_ref[...], (((0,), (0,)), ((), ())),
      preferred_element_type=jnp.float32)

  @pl.when(i == pl.num_programs(0) - 1)
  def _():
    a = acc_ref[...]
    o_ref[...] = a[:, :D] * (1.0 / jnp.maximum(a[:, D:], 1.0))


@jax.jit
def _edge_mm(cmat, xa):
  rb = 1024
  grid = NP // rb
  return pl.pallas_call(
      _edge_body,
      grid=(grid,),
      in_specs=[
          pl.BlockSpec((rb, CP), lambda i: (i, 0)),
          pl.BlockSpec((rb, 2 * D), lambda i: (i, 0)),
      ],
      out_specs=pl.BlockSpec((CP, D), lambda i: (0, 0)),
      out_shape=jax.ShapeDtypeStruct((CP, D), jnp.float32),
      scratch_shapes=[pltpu.VMEM((CP, 2 * D), jnp.float32)],
  )(cmat, xa)


def _node_body(c_ref, ea_ref, x_ref, w_ref, b_ref, o_ref):
  z = jnp.dot(c_ref[...], ea_ref[...], preferred_element_type=jnp.float32)
  nf = z[:, :D] * (1.0 / jnp.maximum(z[:, D:], 1.0))
  y = jnp.dot(nf, w_ref[...], preferred_element_type=jnp.float32) + b_ref[...]
  o_ref[...] = jnp.maximum(y, 0.0) + x_ref[...]


@jax.jit
def _node_mm(cmat, ea, xres, w, b):
  rb = 1024
  grid = NP // rb
  return pl.pallas_call(
      _node_body,
      grid=(grid,),
      in_specs=[
          pl.BlockSpec((rb, CP), lambda i: (i, 0)),
          pl.BlockSpec((CP, 2 * D), lambda i: (0, 0)),
          pl.BlockSpec((rb, D), lambda i: (i, 0)),
          pl.BlockSpec((D, D), lambda i: (0, 0)),
          pl.BlockSpec((1, D), lambda i: (0, 0)),
      ],
      out_specs=pl.BlockSpec((rb, D), lambda i: (i, 0)),
      out_shape=jax.ShapeDtypeStruct((NP, D), jnp.float32),
  )(cmat, ea, xres, w, b.reshape(1, D))


def _fusion_body(ev_ref, ob_ref, w1_ref, w2_ref, b_ref, o_ref):
  ev = ev_ref[...]
  ob = ob_ref[...]
  z = (jnp.dot(ob, w1_ref[...], preferred_element_type=jnp.float32)
       + jnp.dot(ev, w2_ref[...], preferred_element_type=jnp.float32)
       + b_ref[...])
  g = jax.nn.sigmoid(z)
  o_ref[...] = g * ob + (1.0 - g) * ev


@jax.jit
def _fusion(ev, ob, w1, w2, b):
  n = ev.shape[0]
  return pl.pallas_call(
      _fusion_body,
      grid=(1,),
      in_specs=[
          pl.BlockSpec((n, D), lambda i: (0, 0)),
          pl.BlockSpec((n, D), lambda i: (0, 0)),
          pl.BlockSpec((D, D), lambda i: (0, 0)),
          pl.BlockSpec((D, D), lambda i: (0, 0)),
          pl.BlockSpec((1, D), lambda i: (0, 0)),
      ],
      out_specs=pl.BlockSpec((n, D), lambda i: (0, 0)),
      out_shape=jax.ShapeDtypeStruct((n, D), jnp.float32),
  )(ev, ob, w1, w2, b.reshape(1, D))


def kernel(object_X, event_X, W_ev, b_ev, g_ev, be_ev, W_ob, b_ob, g_ob, be_ob,
           W1, b1, W2, b2, Wg, bg, node_idx, hedge_idx, main_object, event_sel):
  flat = (node_idx * CP + hedge_idx).reshape(NS, IPT)
  c1d = _sc_build(flat)
  cmat = c1d.reshape(NP, CP)

  ev = _proj(event_X, W_ev, b_ev, g_ev, be_ev)
  ob = _proj(object_X, W_ob, b_ob, g_ob, be_ob)
  X = jnp.concatenate([ev, ob, jnp.zeros((NP - N_NODES, D), jnp.float32)],
                      axis=0)

  ones_np = jnp.ones((NP, D), jnp.float32)
  ones_cp = jnp.ones((CP, D), jnp.float32)

  ef1 = _edge_mm(cmat, jnp.concatenate([X, ones_np], axis=1))
  H1 = _node_mm(cmat, jnp.concatenate([ef1, ones_cp], axis=1), X, W1, b1)
  ef2 = _edge_mm(cmat, jnp.concatenate([H1, ones_np], axis=1))
  H2 = _node_mm(cmat, jnp.concatenate([ef2, ones_cp], axis=1), H1, W2, b2)

  sel = jnp.concatenate([event_sel, main_object + N_EVENTS], axis=0)
  rows = _sc_gather(H2, sel)
  return _fusion(rows[:1024], rows[1024:], Wg[:D], Wg[D:], bg)


# speedup vs baseline: 3.7071x; 1.2284x over previous
import functools

import jax
import jax.numpy as jnp
from jax import lax
from jax.experimental import pallas as pl
from jax.experimental.pallas import tpu as pltpu
from jax.experimental.pallas import tpu_sc as plsc

N_EVENTS = 6000
N_OBJECTS = 4000
N_NODES = 10000
N_HEDGES = 2000
N_INC = 320000
D = 128

NC = 2
NS = 16
NW = NC * NS

NP = 10240
CP = 2048
RB = 640
SW = NP // (NC * RB)
ROWS_SC = RB * SW
IPT = N_INC // NS
CS = 4000
RN = RB * CP
TILE_ELS = RN // NS

_mesh = lambda: plsc.VectorSubcoreMesh(
    core_axis_name="c", subcore_axis_name="s", num_cores=NC, num_subcores=NS)

_sc_params = lambda: pltpu.CompilerParams(needs_layout_passes=False)


ZB = 8192
DUMP = 2048


def _build_body(flat, out, flat_v, off_v, ones_v, zero_v, accum):
  c = lax.axis_index("c")
  s = lax.axis_index("s")
  pltpu.sync_copy(flat.at[s], flat_v)
  zeros16 = jnp.zeros((16,), jnp.float32)
  ones16 = jnp.ones((16,), jnp.float32)

  def of(i, _):
    ones_v[pl.ds(i * 16, 16)] = ones16
    return 0
  lax.fori_loop(0, CS // 16, of, 0)

  def zf(i, _):
    zero_v[pl.ds(i * 16, 16)] = zeros16
    return 0
  lax.fori_loop(0, ZB // 16, zf, 0)

  def sweep(k, _):
    base = (c * ROWS_SC + k * RB) * CP

    def z(i, _):
      pltpu.sync_copy(zero_v, accum.at[pl.ds(s * TILE_ELS + i * ZB, ZB)])
      return 0
    lax.fori_loop(0, TILE_ELS // ZB, z, 0)
    plsc.subcore_barrier()

    def chunk(q, _):
      def off_i(i, _):
        fv = flat_v[pl.ds(q * CS + i * 16, 16)]
        off = fv - base
        inb = (off >= 0) & (off < RN)
        off_v[pl.ds(i * 16, 16)] = jnp.where(
            inb, off, RN + (fv & (DUMP - 1)))
        return 0
      lax.fori_loop(0, CS // 16, off_i, 0)
      pltpu.sync_copy(ones_v, accum.at[off_v], add=True)
      return 0
    lax.fori_loop(0, IPT // CS, chunk, 0)
    plsc.subcore_barrier()

    row0 = c * ROWS_SC + k * RB
    pltpu.sync_copy(accum.at[pl.ds(s * TILE_ELS, TILE_ELS)],
                    out.at[pl.ds(row0 * CP + s * TILE_ELS, TILE_ELS)])
    plsc.subcore_barrier()
    return 0
  lax.fori_loop(0, SW, sweep, 0)


@jax.jit
def _sc_build(flat):
  return pl.kernel(
      _build_body,
      out_type=jax.ShapeDtypeStruct((NP * CP,), jnp.float32),
      mesh=_mesh(),
      compiler_params=_sc_params(),
      scratch_types=[
          pltpu.VMEM((IPT,), jnp.int32),
          pltpu.VMEM((CS,), jnp.int32),
          pltpu.VMEM((CS,), jnp.float32),
          pltpu.VMEM((ZB,), jnp.float32),
          pltpu.VMEM_SHARED((RN + DUMP,), jnp.float32),
      ],
  )(flat)


def _gather_body(table, idx, out, idx_v, rows_v, sem):
  c = lax.axis_index("c")
  s = lax.axis_index("s")
  wid = c * NS + s
  bpw = 2048 // NW
  base = wid * bpw
  pltpu.sync_copy(idx.at[pl.ds(base, bpw)], idx_v)
  pltpu.async_copy(table.at[idx_v], rows_v, sem).wait()
  pltpu.sync_copy(rows_v, out.at[pl.ds(base, bpw)])


@jax.jit
def _sc_gather(table, idx):
  bpw = 2048 // NW
  return pl.kernel(
      _gather_body,
      out_type=jax.ShapeDtypeStruct((2048, D), jnp.float32),
      mesh=_mesh(),
      compiler_params=_sc_params(),
      scratch_types=[
          pltpu.VMEM((bpw,), jnp.int32),
          pltpu.VMEM((bpw, D), jnp.float32),
          pltpu.SemaphoreType.DMA,
      ],
  )(table, idx)


def _proj_body(x_ref, w_ref, b_ref, g_ref, be_ref, o_ref):
  y = jnp.dot(x_ref[...], w_ref[...], preferred_element_type=jnp.float32)
  y = y + b_ref[...]
  y = jnp.where(y >= 0, y, 0.2 * y)
  m = jnp.mean(y, axis=-1, keepdims=True)
  v = jnp.mean((y - m) ** 2, axis=-1, keepdims=True)
  o_ref[...] = (y - m) / jnp.sqrt(v + 1e-5) * g_ref[...] + be_ref[...]


@jax.jit
def _proj(x, w, b, g, be):
  n = x.shape[0]
  rb = 1000
  grid = n // rb
  return pl.pallas_call(
      _proj_body,
      grid=(grid,),
      in_specs=[
          pl.BlockSpec((rb, D), lambda i: (i, 0)),
          pl.BlockSpec((D, D), lambda i: (0, 0)),
          pl.BlockSpec((1, D), lambda i: (0, 0)),
          pl.BlockSpec((1, D), lambda i: (0, 0)),
          pl.BlockSpec((1, D), lambda i: (0, 0)),
      ],
      out_specs=pl.BlockSpec((rb, D), lambda i: (i, 0)),
      out_shape=jax.ShapeDtypeStruct((n, D), jnp.float32),
  )(x, w, b.reshape(1, D), g.reshape(1, D), be.reshape(1, D))


def _edge_body(c_ref, xa_ref, o_ref, acc_ref):
  i = pl.program_id(0)

  @pl.when(i == 0)
  def _():
    acc_ref[...] = jnp.zeros_like(acc_ref)

  acc_ref[...] += lax.dot_general(
      c_ref[...], xa_ref[...], (((0,), (0,)), ((), ())),
      preferred_element_type=jnp.float32)

  @pl.when(i == pl.num_programs(0) - 1)
  def _():
    a = acc_ref[...]
    o_ref[...] = a[:, :D] * (1.0 / jnp.maximum(a[:, D:], 1.0))


@jax.jit
def _edge_mm(cmat, xa):
  rb = 1024
  grid = NP // rb
  return pl.pallas_call(
      _edge_body,
      grid=(grid,),
      in_specs=[
          pl.BlockSpec((rb, CP), lambda i: (i, 0)),
          pl.BlockSpec((rb, 2 * D), lambda i: (i, 0)),
      ],
      out_specs=pl.BlockSpec((CP, D), lambda i: (0, 0)),
      out_shape=jax.ShapeDtypeStruct((CP, D), jnp.float32),
      scratch_shapes=[pltpu.VMEM((CP, 2 * D), jnp.float32)],
  )(cmat, xa)


def _node_body(c_ref, ea_ref, x_ref, w_ref, b_ref, o_ref):
  z = jnp.dot(c_ref[...], ea_ref[...], preferred_element_type=jnp.float32)
  nf = z[:, :D] * (1.0 / jnp.maximum(z[:, D:], 1.0))
  y = jnp.dot(nf, w_ref[...], preferred_element_type=jnp.float32) + b_ref[...]
  o_ref[...] = jnp.maximum(y, 0.0) + x_ref[...]


@jax.jit
def _node_mm(cmat, ea, xres, w, b):
  rb = 1024
  grid = NP // rb
  return pl.pallas_call(
      _node_body,
      grid=(grid,),
      in_specs=[
          pl.BlockSpec((rb, CP), lambda i: (i, 0)),
          pl.BlockSpec((CP, 2 * D), lambda i: (0, 0)),
          pl.BlockSpec((rb, D), lambda i: (i, 0)),
          pl.BlockSpec((D, D), lambda i: (0, 0)),
          pl.BlockSpec((1, D), lambda i: (0, 0)),
      ],
      out_specs=pl.BlockSpec((rb, D), lambda i: (i, 0)),
      out_shape=jax.ShapeDtypeStruct((NP, D), jnp.float32),
  )(cmat, ea, xres, w, b.reshape(1, D))


def _fusion_body(ev_ref, ob_ref, w1_ref, w2_ref, b_ref, o_ref):
  ev = ev_ref[...]
  ob = ob_ref[...]
  z = (jnp.dot(ob, w1_ref[...], preferred_element_type=jnp.float32)
       + jnp.dot(ev, w2_ref[...], preferred_element_type=jnp.float32)
       + b_ref[...])
  g = jax.nn.sigmoid(z)
  o_ref[...] = g * ob + (1.0 - g) * ev


@jax.jit
def _fusion(ev, ob, w1, w2, b):
  n = ev.shape[0]
  return pl.pallas_call(
      _fusion_body,
      grid=(1,),
      in_specs=[
          pl.BlockSpec((n, D), lambda i: (0, 0)),
          pl.BlockSpec((n, D), lambda i: (0, 0)),
          pl.BlockSpec((D, D), lambda i: (0, 0)),
          pl.BlockSpec((D, D), lambda i: (0, 0)),
          pl.BlockSpec((1, D), lambda i: (0, 0)),
      ],
      out_specs=pl.BlockSpec((n, D), lambda i: (0, 0)),
      out_shape=jax.ShapeDtypeStruct((n, D), jnp.float32),
  )(ev, ob, w1, w2, b.reshape(1, D))


def kernel(object_X, event_X, W_ev, b_ev, g_ev, be_ev, W_ob, b_ob, g_ob, be_ob,
           W1, b1, W2, b2, Wg, bg, node_idx, hedge_idx, main_object, event_sel):
  flat = (node_idx * CP + hedge_idx).reshape(NS, IPT)
  c1d = _sc_build(flat)
  cmat = c1d.reshape(NP, CP)

  ev = _proj(event_X, W_ev, b_ev, g_ev, be_ev)
  ob = _proj(object_X, W_ob, b_ob, g_ob, be_ob)
  X = jnp.concatenate([ev, ob, jnp.zeros((NP - N_NODES, D), jnp.float32)],
                      axis=0)

  ones_np = jnp.ones((NP, D), jnp.float32)
  ones_cp = jnp.ones((CP, D), jnp.float32)

  ef1 = _edge_mm(cmat, jnp.concatenate([X, ones_np], axis=1))
  H1 = _node_mm(cmat, jnp.concatenate([ef1, ones_cp], axis=1), X, W1, b1)
  ef2 = _edge_mm(cmat, jnp.concatenate([H1, ones_np], axis=1))
  H2 = _node_mm(cmat, jnp.concatenate([ef2, ones_cp], axis=1), H1, W2, b2)

  sel = jnp.concatenate([event_sel, main_object + N_EVENTS], axis=0)
  rows = _sc_gather(H2, sel)
  return _fusion(rows[:1024], rows[1024:], Wg[:D], Wg[D:], bg)
